# baseline (speedup 1.0000x reference)
import functools

import jax
import jax.numpy as jnp
import numpy as np
from jax import lax
from jax.experimental import pallas as pl
from jax.experimental.pallas import tpu as pltpu
from jax.experimental.pallas import tpu_sc as plsc

N_NODES = 10000
N_PAD = 10240
F_QTR = 64
R_BLK = 1000
CHK = 128
WIN = 8
NB = N_NODES // R_BLK
EPS = 1e-5

_ILV = np.arange(16)
_PERM = np.concatenate(
    [np.stack([_ILV + B, _ILV + 16 + B], axis=1).ravel() for B in (0, 32)])

_MESH = dict(core_axis_name="c", subcore_axis_name="s")
NC, NS = 2, 16


def _deg_body(dst_hbm, ew_hbm, out_hbm, db0, db1, eb0, eb1, zbuf, acc,
              ds0, ds1, es0, es1):
    c = lax.axis_index("c")
    s = lax.axis_index("s")

    def zb(i, _):
        zbuf[pl.ds(i * 16, 16)] = jnp.zeros((16,), jnp.float32)
        return 0
    lax.fori_loop(0, 40, zb, 0)
    pltpu.sync_copy(zbuf, acc.at[pl.ds(s * 640, 640)])
    plsc.subcore_barrier()

    nchunks = dst_hbm.shape[2]
    dbufs, ebufs = (db0, db1), (eb0, eb1)
    dsems, esems = (ds0, ds1), (es0, es1)

    pltpu.async_copy(dst_hbm.at[c, s, 0], db0, ds0)
    pltpu.async_copy(ew_hbm.at[c, s, 0], eb0, es0)
    pltpu.async_copy(dst_hbm.at[c, s, 1], db1, ds1)
    pltpu.async_copy(ew_hbm.at[c, s, 1], eb1, es1)

    def pair(k, _):
        for b in range(2):
            j = 2 * k + b
            pltpu.make_async_copy(dst_hbm.at[c, s, 0], dbufs[b],
                                  dsems[b]).wait()
            pltpu.make_async_copy(ew_hbm.at[c, s, 0], ebufs[b],
                                  esems[b]).wait()
            pltpu.sync_copy(ebufs[b], acc.at[dbufs[b]], add=True)
            nxt = jnp.minimum(j + 2, nchunks - 1)
            pltpu.async_copy(dst_hbm.at[c, s, nxt], dbufs[b], dsems[b])
            pltpu.async_copy(ew_hbm.at[c, s, nxt], ebufs[b], esems[b])
        return 0
    lax.fori_loop(0, nchunks // 2, pair, 0)
    for b in range(2):
        pltpu.make_async_copy(dst_hbm.at[c, s, 0], dbufs[b], dsems[b]).wait()
        pltpu.make_async_copy(ew_hbm.at[c, s, 0], ebufs[b], esems[b]).wait()
    plsc.subcore_barrier()
    pltpu.sync_copy(acc.at[pl.ds(s * 640, 640)],
                    out_hbm.at[pl.ds(c * N_PAD + s * 640, 640)])


def _make_deg(nchunks):
    return functools.partial(
        pl.kernel,
        out_type=jax.ShapeDtypeStruct((NC * N_PAD,), jnp.float32),
        mesh=plsc.VectorSubcoreMesh(**_MESH),
        compiler_params=pltpu.CompilerParams(use_tc_tiling_on_sc=False),
        scratch_types=[
            pltpu.VMEM((128,), jnp.int32),
            pltpu.VMEM((128,), jnp.int32),
            pltpu.VMEM((128,), jnp.float32),
            pltpu.VMEM((128,), jnp.float32),
            pltpu.VMEM((640,), jnp.float32),
            pltpu.VMEM_SHARED((N_PAD,), jnp.float32),
            pltpu.SemaphoreType.DMA,
            pltpu.SemaphoreType.DMA,
            pltpu.SemaphoreType.DMA,
            pltpu.SemaphoreType.DMA,
        ],
    )(_deg_body)


def _agg_body(g_hbm, src_hbm, dst_hbm, ew_hbm, out_hbm,
              sw0, sw1, dw0, dw1, eww0, eww1, gb0, gb1, sb0, sb1, tbl, acc,
              gs0, gs1, ss0, ss1, ws0, ws1):
    c = lax.axis_index("c")
    s = lax.axis_index("s")
    npieces = src_hbm.shape[1] // WIN
    sws, dws, ews_ = (sw0, sw1), (dw0, dw1), (eww0, eww1)
    gbufs, sbufs = (gb0, gb1), (sb0, sb1)
    gsems, ssems = (gs0, gs1), (ss0, ss1)
    wsems = (ws0, ws1)

    def load_win(piece, h):
        sl = pl.ds(piece * WIN, WIN)
        pltpu.async_copy(src_hbm.at[s, sl], sws[h], wsems[h])
        pltpu.async_copy(dst_hbm.at[s, sl], dws[h], wsems[h])
        pltpu.async_copy(ew_hbm.at[s, sl], ews_[h], wsems[h])

    def wait_win(h):
        sl = pl.ds(0, WIN)
        pltpu.make_async_copy(src_hbm.at[s, sl], sws[h], wsems[h]).wait()
        pltpu.make_async_copy(dst_hbm.at[s, sl], dws[h], wsems[h]).wait()
        pltpu.make_async_copy(ew_hbm.at[s, sl], ews_[h], wsems[h]).wait()

    def scale(ewb, t, gb, sb):
        def grp(gi, _2):
            wv = ewb[t, pl.ds(gi * 16, 16)]
            e0 = gi * 16
            for l in range(16):
                w = wv[l]
                for f2 in range(2):
                    hv = gb[e0 + l, pl.ds(32 * f2, 32)]
                    va, vb = plsc.unpack(
                        hv, format=plsc.PackFormat.INTERLEAVED)
                    sb[e0 + l, pl.ds(32 * f2, 16)] = va * w
                    sb[e0 + l, pl.ds(32 * f2 + 16, 16)] = vb * w
            return 0
        lax.fori_loop(0, CHK // 16, grp, 0)

    t0 = jnp.where(s < NS - 1, s * 640, N_NODES - 640)

    for p in range(2):
        q = 2 * c + p
        pltpu.sync_copy(g_hbm.at[pl.ds(q * N_NODES + t0, 640)],
                        tbl.at[pl.ds(t0, 640)])

        def zb(r, _):
            for f in range(4):
                sb0[r, pl.ds(f * 16, 16)] = jnp.zeros((16,), jnp.float32)
            return 0
        lax.fori_loop(0, CHK, zb, 0)
        for k in range(10):
            pltpu.sync_copy(sb0.at[pl.ds(0, 64)],
                            acc.at[pl.ds(s * 640 + k * 64, 64)])

        load_win(0, 0)
        load_win(1, 1)
        wait_win(0)
        plsc.subcore_barrier()
        pltpu.async_copy(tbl.at[sw0.at[0]], gb0, gs0)
        pltpu.async_copy(tbl.at[sw0.at[1]], gb1, gs1)

        def piece_pair(u, _):
            for h in range(2):
                sw, dw, ewb = sws[h], dws[h], ews_[h]
                swn = sws[1 - h]
                for t in range(WIN):
                    b = t % 2
                    gb, sb = gbufs[b], sbufs[b]
                    pltpu.make_async_copy(tbl.at[sw.at[0]], gb,
                                          gsems[b]).wait()
                    if t >= 2:
                        pltpu.make_async_copy(sb, acc.at[dw.at[0]],
                                              ssems[b]).wait()
                    scale(ewb, t, gb, sb)
                    if t < WIN - 2:
                        pltpu.async_copy(tbl.at[sw.at[t + 2]], gb, gsems[b])
                    else:
                        pltpu.async_copy(tbl.at[swn.at[t - (WIN - 2)]], gb,
                                         gsems[b])
                    pltpu.async_copy(sb, acc.at[dw.at[t]], ssems[b],
                                     add=True)
                    if t == WIN - 3:
                        wait_win(1 - h)
                for b in range(2):
                    pltpu.make_async_copy(sbufs[b], acc.at[dw.at[0]],
                                          ssems[b]).wait()
                nxt = jnp.minimum(2 * u + h + 2, npieces - 1)
                load_win(nxt, h)
            return 0
        lax.fori_loop(0, npieces // 2, piece_pair, 0)

        for b in range(2):
            pltpu.make_async_copy(tbl.at[sw0.at[0]], gbufs[b],
                                  gsems[b]).wait()
        wait_win(1)
        plsc.subcore_barrier()

        def dr(k, _):
            pltpu.sync_copy(
                acc.at[pl.ds(s * 640 + k * 64, 64)],
                out_hbm.at[pl.ds(q * N_PAD + s * 640 + k * 64, 64)])
            return 0
        lax.fori_loop(0, 10, dr, 0)


def _make_agg(nchunks):
    return functools.partial(
        pl.kernel,
        out_type=jax.ShapeDtypeStruct((4 * N_PAD, F_QTR), jnp.float32),
        mesh=plsc.VectorSubcoreMesh(**_MESH),
        compiler_params=pltpu.CompilerParams(use_tc_tiling_on_sc=False,
                                             needs_layout_passes=False),
        scratch_types=[
            pltpu.VMEM((WIN, CHK), jnp.int32),
            pltpu.VMEM((WIN, CHK), jnp.int32),
            pltpu.VMEM((WIN, CHK), jnp.int32),
            pltpu.VMEM((WIN, CHK), jnp.int32),
            pltpu.VMEM((WIN, CHK), jnp.float32),
            pltpu.VMEM((WIN, CHK), jnp.float32),
            pltpu.VMEM((CHK, F_QTR), jnp.bfloat16),
            pltpu.VMEM((CHK, F_QTR), jnp.bfloat16),
            pltpu.VMEM((CHK, F_QTR), jnp.float32),
            pltpu.VMEM((CHK, F_QTR), jnp.float32),
            pltpu.VMEM_SHARED((N_NODES, F_QTR), jnp.bfloat16),
            pltpu.VMEM_SHARED((N_PAD, F_QTR), jnp.float32),
            pltpu.SemaphoreType.DMA,
            pltpu.SemaphoreType.DMA,
            pltpu.SemaphoreType.DMA,
            pltpu.SemaphoreType.DMA,
            pltpu.SemaphoreType.DMA,
            pltpu.SemaphoreType.DMA,
        ],
    )(_agg_body)


def _mm1_body(x_ref, w_ref, da_ref, db_ref, o_ref):
    dinv = lax.rsqrt(da_ref[...] + db_ref[...] + 1.0)
    h = lax.dot_general(x_ref[...], w_ref[...], (((1,), (1,)), ((), ())),
                        precision=lax.Precision.HIGHEST,
                        preferred_element_type=jnp.float32)
    o_ref[...] = h * dinv


def _tc_first(x, W1, dega, degb):
    return pl.pallas_call(
        _mm1_body,
        grid=(4, NB),
        in_specs=[
            pl.BlockSpec((R_BLK, 128), lambda j, i: (i, 0)),
            pl.BlockSpec((F_QTR, 128), lambda j, i: (j, 0)),
            pl.BlockSpec((R_BLK, 1), lambda j, i: (i, 0)),
            pl.BlockSpec((R_BLK, 1), lambda j, i: (i, 0)),
        ],
        out_specs=pl.BlockSpec((R_BLK, F_QTR), lambda j, i: (j * NB + i, 0)),
        out_shape=jax.ShapeDtypeStruct((4 * N_NODES, F_QTR), jnp.float32),
    )(x, W1, dega, degb)


def _zcat(aq, gq, dinv, b, s, t):
    zs = []
    for q in range(4):
        pre = (aq[q][...] + gq[q][...]) * dinv + b[:, q * 64:(q + 1) * 64]
        zs.append(jnp.maximum(pre, 0.0))
    return jnp.concatenate(zs, axis=1) * s + t


def _mid_body(a0, a1, a2, a3, g0, g1, g2, g3, da, db,
              b_ref, bw, bb, brm, brv, w2_ref, o_ref):
    dinv = lax.rsqrt(da[...] + db[...] + 1.0)
    s = bw[...] / jnp.sqrt(brv[...] + EPS)
    t = bb[...] - brm[...] * s
    z = _zcat((a0, a1, a2, a3), (g0, g1, g2, g3), dinv, b_ref[...], s, t)
    h = lax.dot_general(z, w2_ref[...], (((1,), (1,)), ((), ())),
                        precision=lax.Precision.HIGHEST,
                        preferred_element_type=jnp.float32)
    o_ref[...] = h * dinv


def _tc_mid(accq, g, dega, degb, b1, bn_w, bn_b, bn_rm, bn_rv, W2):
    row = lambda j, i: (i, 0)
    vec = lambda j, i: (0, 0)
    gq = lambda q: (lambda j, i, q=q: (q * NB + i, 0))
    return pl.pallas_call(
        _mid_body,
        grid=(4, NB),
        in_specs=[
            pl.BlockSpec((R_BLK, F_QTR), row),
            pl.BlockSpec((R_BLK, F_QTR), row),
            pl.BlockSpec((R_BLK, F_QTR), row),
            pl.BlockSpec((R_BLK, F_QTR), row),
            pl.BlockSpec((R_BLK, F_QTR), gq(0)),
            pl.BlockSpec((R_BLK, F_QTR), gq(1)),
            pl.BlockSpec((R_BLK, F_QTR), gq(2)),
            pl.BlockSpec((R_BLK, F_QTR), gq(3)),
            pl.BlockSpec((R_BLK, 1), row),
            pl.BlockSpec((R_BLK, 1), row),
            pl.BlockSpec((1, 256), vec),
            pl.BlockSpec((1, 256), vec),
            pl.BlockSpec((1, 256), vec),
            pl.BlockSpec((1, 256), vec),
            pl.BlockSpec((1, 256), vec),
            pl.BlockSpec((F_QTR, 256), lambda j, i: (j, 0)),
        ],
        out_specs=pl.BlockSpec((R_BLK, F_QTR), lambda j, i: (j * NB + i, 0)),
        out_shape=jax.ShapeDtypeStruct((4 * N_NODES, F_QTR), jnp.float32),
    )(*accq, g, g, g, g, dega, degb, b1[None, :], bn_w[None, :],
      bn_b[None, :], bn_rm[None, :], bn_rv[None, :], W2)


def _last_body(g0, g1, g2, g3, da, db, lw_ref, lb_ref, o_ref):
    rsq = jnp.sqrt(da[...] + db[...] + 1.0)
    z = jnp.concatenate([g0[...], g1[...], g2[...], g3[...]], axis=1) * rsq
    h = lax.dot_general(z, lw_ref[...], (((1,), (1,)), ((), ())),
                        precision=lax.Precision.HIGHEST,
                        preferred_element_type=jnp.float32)
    o_ref[...] = h + lb_ref[...]


def _tc_last(g, dega, degb, lin_w, lin_b):
    row = lambda i: (i, 0)
    vec = lambda i: (0, 0)
    gq = lambda q: (lambda i, q=q: (q * NB + i, 0))
    return pl.pallas_call(
        _last_body,
        grid=(NB,),
        in_specs=[
            pl.BlockSpec((R_BLK, F_QTR), gq(0)),
            pl.BlockSpec((R_BLK, F_QTR), gq(1)),
            pl.BlockSpec((R_BLK, F_QTR), gq(2)),
            pl.BlockSpec((R_BLK, F_QTR), gq(3)),
            pl.BlockSpec((R_BLK, 1), row),
            pl.BlockSpec((R_BLK, 1), row),
            pl.BlockSpec((64, 256), vec),
            pl.BlockSpec((1, 64), vec),
        ],
        out_specs=pl.BlockSpec((R_BLK, 64), row),
        out_shape=jax.ShapeDtypeStruct((N_NODES, 64), jnp.float32),
    )(g, g, g, g, dega, degb, lin_w, lin_b[None, :])


def _pad_edges(src, dst, ew, granule):
    e = src.shape[0]
    e_pad = ((e + granule - 1) // granule) * granule
    pad = e_pad - e
    if pad:
        fill = (jnp.arange(pad, dtype=jnp.int32) * 37) % N_NODES
        src = jnp.concatenate([src, fill])
        dst = jnp.concatenate([dst, fill])
        ew = jnp.concatenate([ew, jnp.zeros((pad,), ew.dtype)])
    return src, dst, ew, e_pad


def _quarters(accp):
    return tuple(accp[q * N_PAD:q * N_PAD + N_NODES] for q in range(4))


def kernel(x, edge_index, edge_weight, W1, b1, W2, b2, lin_w, lin_b,
           bn1_w, bn1_b, bn1_rm, bn1_rv, bn2_w, bn2_b, bn2_rm, bn2_rv):
    src = edge_index[0].astype(jnp.int32)
    dst = edge_index[1].astype(jnp.int32)
    ew = edge_weight.astype(jnp.float32)

    sD, dD, wD, epD = _pad_edges(src, dst, ew, NC * NS * 256)
    cD = epD // (NC * NS * 128)
    dstD = dD.reshape(NC, NS, cD, 128)
    ewD = wD.reshape(NC, NS, cD, 128)

    sA, dA, wA, epA = _pad_edges(src, dst, ew, NS * CHK * 2 * WIN)
    cA = epA // (NS * CHK)
    src3 = sA.reshape(NS, cA, CHK)
    dst3 = dA.reshape(NS, cA, CHK)
    ew3 = wA.reshape(NS, cA, CHK)

    degp = _make_deg(cD)(dstD, ewD)
    dega = degp[:N_NODES, None]
    degb = degp[N_PAD:N_PAD + N_NODES, None]

    agg = _make_agg(cA)

    g1 = _tc_first(x, W1, dega, degb)

    eye = jnp.eye(W2.shape[0], dtype=jnp.float32)
    xs = (jnp.stack([W2, eye]), jnp.stack([b1, b2]),
          jnp.stack([bn1_w, bn2_w]), jnp.stack([bn1_b, bn2_b]),
          jnp.stack([bn1_rm, bn2_rm]), jnp.stack([bn1_rv, bn2_rv]))

    def body(g, x_l):
        Wl, bl, bwl, bbl, brml, brvl = x_l
        gbf = g[:, _PERM].astype(jnp.bfloat16)
        accp = agg(gbf, src3, dst3, ew3)
        g_next = _tc_mid(_quarters(accp), g, dega, degb, bl,
                         bwl, bbl, brml, brvl, Wl)
        return g_next, None

    gz, _ = lax.scan(body, g1, xs)
    return _tc_last(gz, dega, degb, lin_w, lin_b)


# speedup vs baseline: 1.4773x; 1.4773x over previous
import functools

import jax
import jax.numpy as jnp
from jax import lax
from jax.experimental import pallas as pl
from jax.experimental.pallas import tpu as pltpu
from jax.experimental.pallas import tpu_sc as plsc

N_NODES = 10000
N_PAD = 10240
F_QTR = 64
R_BLK = 1000
CHK = 128
WIN = 8
NB = N_NODES // R_BLK
EPS = 1e-5

_MESH = dict(core_axis_name="c", subcore_axis_name="s")
NC, NS = 2, 16


def _deg_body(dst_hbm, ew_hbm, out_hbm, db0, db1, eb0, eb1, zbuf, acc,
              ds0, ds1, es0, es1):
    c = lax.axis_index("c")
    s = lax.axis_index("s")

    def zb(i, _):
        zbuf[pl.ds(i * 16, 16)] = jnp.zeros((16,), jnp.float32)
        return 0
    lax.fori_loop(0, 40, zb, 0)
    pltpu.sync_copy(zbuf, acc.at[pl.ds(s * 640, 640)])
    plsc.subcore_barrier()

    nchunks = dst_hbm.shape[2]
    dbufs, ebufs = (db0, db1), (eb0, eb1)
    dsems, esems = (ds0, ds1), (es0, es1)

    pltpu.async_copy(dst_hbm.at[c, s, 0], db0, ds0)
    pltpu.async_copy(ew_hbm.at[c, s, 0], eb0, es0)
    pltpu.async_copy(dst_hbm.at[c, s, 1], db1, ds1)
    pltpu.async_copy(ew_hbm.at[c, s, 1], eb1, es1)

    def pair(k, _):
        for b in range(2):
            j = 2 * k + b
            pltpu.make_async_copy(dst_hbm.at[c, s, 0], dbufs[b],
                                  dsems[b]).wait()
            pltpu.make_async_copy(ew_hbm.at[c, s, 0], ebufs[b],
                                  esems[b]).wait()
            pltpu.sync_copy(ebufs[b], acc.at[dbufs[b]], add=True)
            nxt = jnp.minimum(j + 2, nchunks - 1)
            pltpu.async_copy(dst_hbm.at[c, s, nxt], dbufs[b], dsems[b])
            pltpu.async_copy(ew_hbm.at[c, s, nxt], ebufs[b], esems[b])
        return 0
    lax.fori_loop(0, nchunks // 2, pair, 0)
    for b in range(2):
        pltpu.make_async_copy(dst_hbm.at[c, s, 0], dbufs[b], dsems[b]).wait()
        pltpu.make_async_copy(ew_hbm.at[c, s, 0], ebufs[b], esems[b]).wait()
    plsc.subcore_barrier()
    pltpu.sync_copy(acc.at[pl.ds(s * 640, 640)],
                    out_hbm.at[pl.ds(c * N_PAD + s * 640, 640)])


def _make_deg(nchunks):
    return functools.partial(
        pl.kernel,
        out_type=jax.ShapeDtypeStruct((NC * N_PAD,), jnp.float32),
        mesh=plsc.VectorSubcoreMesh(**_MESH),
        compiler_params=pltpu.CompilerParams(use_tc_tiling_on_sc=False),
        scratch_types=[
            pltpu.VMEM((128,), jnp.int32),
            pltpu.VMEM((128,), jnp.int32),
            pltpu.VMEM((128,), jnp.float32),
            pltpu.VMEM((128,), jnp.float32),
            pltpu.VMEM((640,), jnp.float32),
            pltpu.VMEM_SHARED((N_PAD,), jnp.float32),
            pltpu.SemaphoreType.DMA,
            pltpu.SemaphoreType.DMA,
            pltpu.SemaphoreType.DMA,
            pltpu.SemaphoreType.DMA,
        ],
    )(_deg_body)


def _agg_body(g_hbm, src_hbm, dst_hbm, ew_hbm, out_hbm,
              sw0, sw1, dw0, dw1, eww0, eww1, gb0, gb1, sb0, sb1, tbl, acc,
              gs0, gs1, ss0, ss1, ws0, ws1):
    c = lax.axis_index("c")
    s = lax.axis_index("s")
    npieces = src_hbm.shape[1] // WIN
    sws, dws, ews_ = (sw0, sw1), (dw0, dw1), (eww0, eww1)
    gbufs, sbufs = (gb0, gb1), (sb0, sb1)
    gsems, ssems = (gs0, gs1), (ss0, ss1)
    wsems = (ws0, ws1)

    def load_win(piece, h):
        sl = pl.ds(piece * WIN, WIN)
        pltpu.async_copy(src_hbm.at[s, sl], sws[h], wsems[h])
        pltpu.async_copy(dst_hbm.at[s, sl], dws[h], wsems[h])
        pltpu.async_copy(ew_hbm.at[s, sl], ews_[h], wsems[h])

    def wait_win(h):
        sl = pl.ds(0, WIN)
        pltpu.make_async_copy(src_hbm.at[s, sl], sws[h], wsems[h]).wait()
        pltpu.make_async_copy(dst_hbm.at[s, sl], dws[h], wsems[h]).wait()
        pltpu.make_async_copy(ew_hbm.at[s, sl], ews_[h], wsems[h]).wait()

    def scale(ewb, t, gb, sb):
        def grp(gi, _2):
            wv = ewb[t, pl.ds(gi * 16, 16)]
            e0 = gi * 16
            for l in range(16):
                w = wv[l]
                for f in range(4):
                    slf = pl.ds(f * 16, 16)
                    sb[e0 + l, slf] = gb[e0 + l, slf] * w
            return 0
        lax.fori_loop(0, CHK // 16, grp, 0)

    t0 = jnp.where(s < NS - 1, s * 640, N_NODES - 640)

    for p in range(2):
        q = 2 * c + p
        pltpu.sync_copy(g_hbm.at[pl.ds(q * N_NODES + t0, 640)],
                        tbl.at[pl.ds(t0, 640)])

        def zb(r, _):
            for f in range(4):
                gb0[r, pl.ds(f * 16, 16)] = jnp.zeros((16,), jnp.float32)
            return 0
        lax.fori_loop(0, CHK, zb, 0)
        for k in range(10):
            pltpu.sync_copy(gb0.at[pl.ds(0, 64)],
                            acc.at[pl.ds(s * 640 + k * 64, 64)])

        load_win(0, 0)
        load_win(1, 1)
        wait_win(0)
        plsc.subcore_barrier()
        pltpu.async_copy(tbl.at[sw0.at[0]], gb0, gs0)
        pltpu.async_copy(tbl.at[sw0.at[1]], gb1, gs1)

        def piece_pair(u, _):
            for h in range(2):
                sw, dw, ewb = sws[h], dws[h], ews_[h]
                swn = sws[1 - h]
                for t in range(WIN):
                    b = t % 2
                    gb, sb = gbufs[b], sbufs[b]
                    pltpu.make_async_copy(tbl.at[sw.at[0]], gb,
                                          gsems[b]).wait()
                    if t >= 2:
                        pltpu.make_async_copy(sb, acc.at[dw.at[0]],
                                              ssems[b]).wait()
                    scale(ewb, t, gb, sb)
                    if t < WIN - 2:
                        pltpu.async_copy(tbl.at[sw.at[t + 2]], gb, gsems[b])
                    else:
                        pltpu.async_copy(tbl.at[swn.at[t - (WIN - 2)]], gb,
                                         gsems[b])
                    pltpu.async_copy(sb, acc.at[dw.at[t]], ssems[b],
                                     add=True)
                    if t == WIN - 3:
                        wait_win(1 - h)
                for b in range(2):
                    pltpu.make_async_copy(sbufs[b], acc.at[dw.at[0]],
                                          ssems[b]).wait()
                nxt = jnp.minimum(2 * u + h + 2, npieces - 1)
                load_win(nxt, h)
            return 0
        lax.fori_loop(0, npieces // 2, piece_pair, 0)

        for b in range(2):
            pltpu.make_async_copy(tbl.at[sw0.at[0]], gbufs[b],
                                  gsems[b]).wait()
        wait_win(1)
        plsc.subcore_barrier()

        def dr(k, _):
            pltpu.sync_copy(
                acc.at[pl.ds(s * 640 + k * 64, 64)],
                out_hbm.at[pl.ds(q * N_PAD + s * 640 + k * 64, 64)])
            return 0
        lax.fori_loop(0, 10, dr, 0)


def _make_agg(nchunks):
    return functools.partial(
        pl.kernel,
        out_type=jax.ShapeDtypeStruct((4 * N_PAD, F_QTR), jnp.float32),
        mesh=plsc.VectorSubcoreMesh(**_MESH),
        compiler_params=pltpu.CompilerParams(use_tc_tiling_on_sc=False),
        scratch_types=[
            pltpu.VMEM((WIN, CHK), jnp.int32),
            pltpu.VMEM((WIN, CHK), jnp.int32),
            pltpu.VMEM((WIN, CHK), jnp.int32),
            pltpu.VMEM((WIN, CHK), jnp.int32),
            pltpu.VMEM((WIN, CHK), jnp.float32),
            pltpu.VMEM((WIN, CHK), jnp.float32),
            pltpu.VMEM((CHK, F_QTR), jnp.float32),
            pltpu.VMEM((CHK, F_QTR), jnp.float32),
            pltpu.VMEM((CHK, F_QTR), jnp.float32),
            pltpu.VMEM((CHK, F_QTR), jnp.float32),
            pltpu.VMEM_SHARED((N_NODES, F_QTR), jnp.float32),
            pltpu.VMEM_SHARED((N_PAD, F_QTR), jnp.float32),
            pltpu.SemaphoreType.DMA,
            pltpu.SemaphoreType.DMA,
            pltpu.SemaphoreType.DMA,
            pltpu.SemaphoreType.DMA,
            pltpu.SemaphoreType.DMA,
            pltpu.SemaphoreType.DMA,
        ],
    )(_agg_body)


def _mm1_body(x_ref, w_ref, da_ref, db_ref, o_ref):
    dinv = lax.rsqrt(da_ref[...] + db_ref[...] + 1.0)
    h = lax.dot_general(x_ref[...], w_ref[...], (((1,), (1,)), ((), ())),
                        precision=lax.Precision.HIGHEST,
                        preferred_element_type=jnp.float32)
    o_ref[...] = h * dinv


def _tc_first(x, W1, dega, degb):
    return pl.pallas_call(
        _mm1_body,
        grid=(4, NB),
        in_specs=[
            pl.BlockSpec((R_BLK, 128), lambda j, i: (i, 0)),
            pl.BlockSpec((F_QTR, 128), lambda j, i: (j, 0)),
            pl.BlockSpec((R_BLK, 1), lambda j, i: (i, 0)),
            pl.BlockSpec((R_BLK, 1), lambda j, i: (i, 0)),
        ],
        out_specs=pl.BlockSpec((R_BLK, F_QTR), lambda j, i: (j * NB + i, 0)),
        out_shape=jax.ShapeDtypeStruct((4 * N_NODES, F_QTR), jnp.float32),
    )(x, W1, dega, degb)


def _zcat(aq, gq, dinv, b, s, t):
    zs = []
    for q in range(4):
        pre = (aq[q][...] + gq[q][...]) * dinv + b[:, q * 64:(q + 1) * 64]
        zs.append(jnp.maximum(pre, 0.0))
    return jnp.concatenate(zs, axis=1) * s + t


def _mid_body(a0, a1, a2, a3, g0, g1, g2, g3, da, db,
              b_ref, bw, bb, brm, brv, w2_ref, o_ref):
    dinv = lax.rsqrt(da[...] + db[...] + 1.0)
    s = bw[...] / jnp.sqrt(brv[...] + EPS)
    t = bb[...] - brm[...] * s
    z = _zcat((a0, a1, a2, a3), (g0, g1, g2, g3), dinv, b_ref[...], s, t)
    h = lax.dot_general(z, w2_ref[...], (((1,), (1,)), ((), ())),
                        precision=lax.Precision.HIGHEST,
                        preferred_element_type=jnp.float32)
    o_ref[...] = h * dinv


def _tc_mid(accq, g, dega, degb, b1, bn_w, bn_b, bn_rm, bn_rv, W2):
    row = lambda j, i: (i, 0)
    vec = lambda j, i: (0, 0)
    gq = lambda q: (lambda j, i, q=q: (q * NB + i, 0))
    return pl.pallas_call(
        _mid_body,
        grid=(4, NB),
        in_specs=[
            pl.BlockSpec((R_BLK, F_QTR), row),
            pl.BlockSpec((R_BLK, F_QTR), row),
            pl.BlockSpec((R_BLK, F_QTR), row),
            pl.BlockSpec((R_BLK, F_QTR), row),
            pl.BlockSpec((R_BLK, F_QTR), gq(0)),
            pl.BlockSpec((R_BLK, F_QTR), gq(1)),
            pl.BlockSpec((R_BLK, F_QTR), gq(2)),
            pl.BlockSpec((R_BLK, F_QTR), gq(3)),
            pl.BlockSpec((R_BLK, 1), row),
            pl.BlockSpec((R_BLK, 1), row),
            pl.BlockSpec((1, 256), vec),
            pl.BlockSpec((1, 256), vec),
            pl.BlockSpec((1, 256), vec),
            pl.BlockSpec((1, 256), vec),
            pl.BlockSpec((1, 256), vec),
            pl.BlockSpec((F_QTR, 256), lambda j, i: (j, 0)),
        ],
        out_specs=pl.BlockSpec((R_BLK, F_QTR), lambda j, i: (j * NB + i, 0)),
        out_shape=jax.ShapeDtypeStruct((4 * N_NODES, F_QTR), jnp.float32),
    )(*accq, g, g, g, g, dega, degb, b1[None, :], bn_w[None, :],
      bn_b[None, :], bn_rm[None, :], bn_rv[None, :], W2)


def _last_body(g0, g1, g2, g3, da, db, lw_ref, lb_ref, o_ref):
    rsq = jnp.sqrt(da[...] + db[...] + 1.0)
    z = jnp.concatenate([g0[...], g1[...], g2[...], g3[...]], axis=1) * rsq
    h = lax.dot_general(z, lw_ref[...], (((1,), (1,)), ((), ())),
                        precision=lax.Precision.HIGHEST,
                        preferred_element_type=jnp.float32)
    o_ref[...] = h + lb_ref[...]


def _tc_last(g, dega, degb, lin_w, lin_b):
    row = lambda i: (i, 0)
    vec = lambda i: (0, 0)
    gq = lambda q: (lambda i, q=q: (q * NB + i, 0))
    return pl.pallas_call(
        _last_body,
        grid=(NB,),
        in_specs=[
            pl.BlockSpec((R_BLK, F_QTR), gq(0)),
            pl.BlockSpec((R_BLK, F_QTR), gq(1)),
            pl.BlockSpec((R_BLK, F_QTR), gq(2)),
            pl.BlockSpec((R_BLK, F_QTR), gq(3)),
            pl.BlockSpec((R_BLK, 1), row),
            pl.BlockSpec((R_BLK, 1), row),
            pl.BlockSpec((64, 256), vec),
            pl.BlockSpec((1, 64), vec),
        ],
        out_specs=pl.BlockSpec((R_BLK, 64), row),
        out_shape=jax.ShapeDtypeStruct((N_NODES, 64), jnp.float32),
    )(g, g, g, g, dega, degb, lin_w, lin_b[None, :])


def _pad_edges(src, dst, ew, granule):
    e = src.shape[0]
    e_pad = ((e + granule - 1) // granule) * granule
    pad = e_pad - e
    if pad:
        fill = (jnp.arange(pad, dtype=jnp.int32) * 37) % N_NODES
        src = jnp.concatenate([src, fill])
        dst = jnp.concatenate([dst, fill])
        ew = jnp.concatenate([ew, jnp.zeros((pad,), ew.dtype)])
    return src, dst, ew, e_pad


def _quarters(accp):
    return tuple(accp[q * N_PAD:q * N_PAD + N_NODES] for q in range(4))


def kernel(x, edge_index, edge_weight, W1, b1, W2, b2, lin_w, lin_b,
           bn1_w, bn1_b, bn1_rm, bn1_rv, bn2_w, bn2_b, bn2_rm, bn2_rv):
    src = edge_index[0].astype(jnp.int32)
    dst = edge_index[1].astype(jnp.int32)
    ew = edge_weight.astype(jnp.float32)

    sD, dD, wD, epD = _pad_edges(src, dst, ew, NC * NS * 256)
    cD = epD // (NC * NS * 128)
    dstD = dD.reshape(NC, NS, cD, 128)
    ewD = wD.reshape(NC, NS, cD, 128)

    sA, dA, wA, epA = _pad_edges(src, dst, ew, NS * CHK * 2 * WIN)
    cA = epA // (NS * CHK)
    src3 = sA.reshape(NS, cA, CHK)
    dst3 = dA.reshape(NS, cA, CHK)
    ew3 = wA.reshape(NS, cA, CHK)

    degp = _make_deg(cD)(dstD, ewD)
    dega = degp[:N_NODES, None]
    degb = degp[N_PAD:N_PAD + N_NODES, None]

    agg = _make_agg(cA)

    g1 = _tc_first(x, W1, dega, degb)

    eye = jnp.eye(W2.shape[0], dtype=jnp.float32)
    xs = (jnp.stack([W2, eye]), jnp.stack([b1, b2]),
          jnp.stack([bn1_w, bn2_w]), jnp.stack([bn1_b, bn2_b]),
          jnp.stack([bn1_rm, bn2_rm]), jnp.stack([bn1_rv, bn2_rv]))

    def body(g, x_l):
        Wl, bl, bwl, bbl, brml, brvl = x_l
        accp = agg(g, src3, dst3, ew3)
        g_next = _tc_mid(_quarters(accp), g, dega, degb, bl,
                         bwl, bbl, brml, brvl, Wl)
        return g_next, None

    gz, _ = lax.scan(body, g1, xs)
    return _tc_last(gz, dega, degb, lin_w, lin_b)


# speedup vs baseline: 1.7169x; 1.1622x over previous
import functools

import jax
import jax.numpy as jnp
from jax import lax
from jax.experimental import pallas as pl
from jax.experimental.pallas import tpu as pltpu
from jax.experimental.pallas import tpu_sc as plsc

N_NODES = 10000
N_PAD = 10240
F_QTR = 64
R_BLK = 1000
CHK = 128
WIN = 8
NB = N_NODES // R_BLK
EPS = 1e-5

_MESH = dict(core_axis_name="c", subcore_axis_name="s")
NC, NS = 2, 16


def _deg_body(dst_hbm, ew_hbm, out_hbm, db0, db1, eb0, eb1, zbuf, acc,
              ds0, ds1, es0, es1):
    c = lax.axis_index("c")
    s = lax.axis_index("s")

    def zb(i, _):
        zbuf[pl.ds(i * 16, 16)] = jnp.zeros((16,), jnp.float32)
        return 0
    lax.fori_loop(0, 40, zb, 0)
    pltpu.sync_copy(zbuf, acc.at[pl.ds(s * 640, 640)])
    plsc.subcore_barrier()

    nchunks = dst_hbm.shape[2]
    dbufs, ebufs = (db0, db1), (eb0, eb1)
    dsems, esems = (ds0, ds1), (es0, es1)

    pltpu.async_copy(dst_hbm.at[c, s, 0], db0, ds0)
    pltpu.async_copy(ew_hbm.at[c, s, 0], eb0, es0)
    pltpu.async_copy(dst_hbm.at[c, s, 1], db1, ds1)
    pltpu.async_copy(ew_hbm.at[c, s, 1], eb1, es1)

    def pair(k, _):
        for b in range(2):
            j = 2 * k + b
            pltpu.make_async_copy(dst_hbm.at[c, s, 0], dbufs[b],
                                  dsems[b]).wait()
            pltpu.make_async_copy(ew_hbm.at[c, s, 0], ebufs[b],
                                  esems[b]).wait()
            pltpu.sync_copy(ebufs[b], acc.at[dbufs[b]], add=True)
            nxt = jnp.minimum(j + 2, nchunks - 1)
            pltpu.async_copy(dst_hbm.at[c, s, nxt], dbufs[b], dsems[b])
            pltpu.async_copy(ew_hbm.at[c, s, nxt], ebufs[b], esems[b])
        return 0
    lax.fori_loop(0, nchunks // 2, pair, 0)
    for b in range(2):
        pltpu.make_async_copy(dst_hbm.at[c, s, 0], dbufs[b], dsems[b]).wait()
        pltpu.make_async_copy(ew_hbm.at[c, s, 0], ebufs[b], esems[b]).wait()
    plsc.subcore_barrier()
    pltpu.sync_copy(acc.at[pl.ds(s * 640, 640)],
                    out_hbm.at[pl.ds(c * N_PAD + s * 640, 640)])


def _make_deg(nchunks):
    return functools.partial(
        pl.kernel,
        out_type=jax.ShapeDtypeStruct((NC * N_PAD,), jnp.float32),
        mesh=plsc.VectorSubcoreMesh(**_MESH),
        compiler_params=pltpu.CompilerParams(use_tc_tiling_on_sc=False),
        scratch_types=[
            pltpu.VMEM((128,), jnp.int32),
            pltpu.VMEM((128,), jnp.int32),
            pltpu.VMEM((128,), jnp.float32),
            pltpu.VMEM((128,), jnp.float32),
            pltpu.VMEM((640,), jnp.float32),
            pltpu.VMEM_SHARED((N_PAD,), jnp.float32),
            pltpu.SemaphoreType.DMA,
            pltpu.SemaphoreType.DMA,
            pltpu.SemaphoreType.DMA,
            pltpu.SemaphoreType.DMA,
        ],
    )(_deg_body)


def _agg_body(g_hbm, src_hbm, dst_hbm, ew_hbm, out_hbm,
              sw0, sw1, dw0, dw1, eww0, eww1, gb0, gb1, sb0, sb1, tbl, acc,
              gs0, gs1, ss0, ss1, ws0, ws1):
    c = lax.axis_index("c")
    s = lax.axis_index("s")
    npieces = src_hbm.shape[1] // WIN
    sws, dws, ews_ = (sw0, sw1), (dw0, dw1), (eww0, eww1)
    gbufs, sbufs = (gb0, gb1), (sb0, sb1)
    gsems, ssems = (gs0, gs1), (ss0, ss1)
    wsems = (ws0, ws1)

    def load_win(piece, h):
        sl = pl.ds(piece * WIN, WIN)
        pltpu.async_copy(src_hbm.at[s, sl], sws[h], wsems[h])
        pltpu.async_copy(dst_hbm.at[s, sl], dws[h], wsems[h])
        pltpu.async_copy(ew_hbm.at[s, sl], ews_[h], wsems[h])

    def wait_win(h):
        sl = pl.ds(0, WIN)
        pltpu.make_async_copy(src_hbm.at[s, sl], sws[h], wsems[h]).wait()
        pltpu.make_async_copy(dst_hbm.at[s, sl], dws[h], wsems[h]).wait()
        pltpu.make_async_copy(ew_hbm.at[s, sl], ews_[h], wsems[h]).wait()

    def scale(ewb, t, gb, sb):
        def grp(gi, _2):
            wv = ewb[t, pl.ds(gi * 16, 16)]
            e0 = gi * 16
            for l in range(16):
                w = wv[l]
                for f in range(4):
                    slf = pl.ds(f * 16, 16)
                    sb[e0 + l, slf] = gb[e0 + l, slf] * w
            return 0
        lax.fori_loop(0, CHK // 16, grp, 0)

    t0 = jnp.where(s < NS - 1, s * 640, N_NODES - 640)

    for p in range(2):
        q = 2 * c + p
        pltpu.sync_copy(g_hbm.at[q, pl.ds(t0, 640)],
                        tbl.at[pl.ds(t0, 640)])

        def zb(r, _):
            for f in range(4):
                gb0[r, pl.ds(f * 16, 16)] = jnp.zeros((16,), jnp.float32)
            return 0
        lax.fori_loop(0, CHK, zb, 0)
        for k in range(10):
            pltpu.sync_copy(gb0.at[pl.ds(0, 64)],
                            acc.at[pl.ds(s * 640 + k * 64, 64)])

        load_win(0, 0)
        load_win(1, 1)
        wait_win(0)
        plsc.subcore_barrier()
        pltpu.async_copy(tbl.at[sw0.at[0]], gb0, gs0)
        pltpu.async_copy(tbl.at[sw0.at[1]], gb1, gs1)

        def piece_pair(u, _):
            for h in range(2):
                sw, dw, ewb = sws[h], dws[h], ews_[h]
                swn = sws[1 - h]
                for t in range(WIN):
                    b = t % 2
                    gb, sb = gbufs[b], sbufs[b]
                    pltpu.make_async_copy(tbl.at[sw.at[0]], gb,
                                          gsems[b]).wait()
                    if t >= 2:
                        pltpu.make_async_copy(sb, acc.at[dw.at[0]],
                                              ssems[b]).wait()
                    scale(ewb, t, gb, sb)
                    if t < WIN - 2:
                        pltpu.async_copy(tbl.at[sw.at[t + 2]], gb, gsems[b])
                    else:
                        pltpu.async_copy(tbl.at[swn.at[t - (WIN - 2)]], gb,
                                         gsems[b])
                    pltpu.async_copy(sb, acc.at[dw.at[t]], ssems[b],
                                     add=True)
                    if t == WIN - 3:
                        wait_win(1 - h)
                for b in range(2):
                    pltpu.make_async_copy(sbufs[b], acc.at[dw.at[0]],
                                          ssems[b]).wait()
                nxt = jnp.minimum(2 * u + h + 2, npieces - 1)
                load_win(nxt, h)
            return 0
        lax.fori_loop(0, npieces // 2, piece_pair, 0)

        for b in range(2):
            pltpu.make_async_copy(tbl.at[sw0.at[0]], gbufs[b],
                                  gsems[b]).wait()
        wait_win(1)
        plsc.subcore_barrier()

        def dr(k, _):
            pltpu.sync_copy(
                acc.at[pl.ds(s * 640 + k * 64, 64)],
                out_hbm.at[q, pl.ds(s * 640 + k * 64, 64)])
            return 0
        lax.fori_loop(0, 10, dr, 0)


def _make_agg(nchunks):
    return functools.partial(
        pl.kernel,
        out_type=jax.ShapeDtypeStruct((4, N_PAD, F_QTR), jnp.float32),
        mesh=plsc.VectorSubcoreMesh(**_MESH),
        compiler_params=pltpu.CompilerParams(use_tc_tiling_on_sc=False),
        scratch_types=[
            pltpu.VMEM((WIN, CHK), jnp.int32),
            pltpu.VMEM((WIN, CHK), jnp.int32),
            pltpu.VMEM((WIN, CHK), jnp.int32),
            pltpu.VMEM((WIN, CHK), jnp.int32),
            pltpu.VMEM((WIN, CHK), jnp.float32),
            pltpu.VMEM((WIN, CHK), jnp.float32),
            pltpu.VMEM((CHK, F_QTR), jnp.float32),
            pltpu.VMEM((CHK, F_QTR), jnp.float32),
            pltpu.VMEM((CHK, F_QTR), jnp.float32),
            pltpu.VMEM((CHK, F_QTR), jnp.float32),
            pltpu.VMEM_SHARED((N_NODES, F_QTR), jnp.float32),
            pltpu.VMEM_SHARED((N_PAD, F_QTR), jnp.float32),
            pltpu.SemaphoreType.DMA,
            pltpu.SemaphoreType.DMA,
            pltpu.SemaphoreType.DMA,
            pltpu.SemaphoreType.DMA,
            pltpu.SemaphoreType.DMA,
            pltpu.SemaphoreType.DMA,
        ],
    )(_agg_body)


def _mm1_body(x_ref, w_ref, da_ref, db_ref, o_ref):
    dinv = lax.rsqrt(da_ref[...] + db_ref[...] + 1.0)
    h = lax.dot_general(x_ref[...], w_ref[...], (((1,), (1,)), ((), ())),
                        precision=lax.Precision.HIGHEST,
                        preferred_element_type=jnp.float32)
    g = h * dinv
    for q in range(4):
        o_ref[q] = g[:, q * F_QTR:(q + 1) * F_QTR]


def _tc_first(x, W1, dega, degb):
    row = lambda i: (i, 0)
    return pl.pallas_call(
        _mm1_body,
        grid=(NB,),
        in_specs=[
            pl.BlockSpec((R_BLK, 128), row),
            pl.BlockSpec((256, 128), lambda i: (0, 0)),
            pl.BlockSpec((R_BLK, 1), row),
            pl.BlockSpec((R_BLK, 1), row),
        ],
        out_specs=pl.BlockSpec((4, R_BLK, F_QTR), lambda i: (0, i, 0)),
        out_shape=jax.ShapeDtypeStruct((4, N_NODES, F_QTR), jnp.float32),
    )(x, W1, dega, degb)


def _mid_body(a_ref, g_ref, da, db, b_ref, bw, bb, brm, brv, w_ref, o_ref):
    dinv = lax.rsqrt(da[...] + db[...] + 1.0)
    sc = bw[...] / jnp.sqrt(brv[...] + EPS)
    t = bb[...] - brm[...] * sc
    b = b_ref[...]
    a = a_ref[...]
    g = g_ref[...]
    zs = []
    for q in range(4):
        pre = (a[q] + g[q]) * dinv + b[:, q * F_QTR:(q + 1) * F_QTR]
        zs.append(jnp.maximum(pre, 0.0))
    z = jnp.concatenate(zs, axis=1) * sc + t
    w = w_ref[...]
    for q in range(4):
        h = lax.dot_general(z, w[q * F_QTR:(q + 1) * F_QTR, :],
                            (((1,), (1,)), ((), ())),
                            precision=lax.Precision.HIGHEST,
                            preferred_element_type=jnp.float32)
        o_ref[q] = h * dinv


def _tc_mid(acc3, g3, dega, degb, b1, bn_w, bn_b, bn_rm, bn_rv, W2):
    row = lambda i: (i, 0)
    vec = lambda i: (0, 0)
    blk3 = pl.BlockSpec((4, R_BLK, F_QTR), lambda i: (0, i, 0))
    return pl.pallas_call(
        _mid_body,
        grid=(NB,),
        in_specs=[
            blk3,
            blk3,
            pl.BlockSpec((R_BLK, 1), row),
            pl.BlockSpec((R_BLK, 1), row),
            pl.BlockSpec((1, 256), vec),
            pl.BlockSpec((1, 256), vec),
            pl.BlockSpec((1, 256), vec),
            pl.BlockSpec((1, 256), vec),
            pl.BlockSpec((1, 256), vec),
            pl.BlockSpec((256, 256), vec),
        ],
        out_specs=blk3,
        out_shape=jax.ShapeDtypeStruct((4, N_NODES, F_QTR), jnp.float32),
    )(acc3, g3, dega, degb, b1[None, :], bn_w[None, :], bn_b[None, :],
      bn_rm[None, :], bn_rv[None, :], W2)


def _last_body(g_ref, da, db, lw_ref, lb_ref, o_ref):
    rsq = jnp.sqrt(da[...] + db[...] + 1.0)
    g = g_ref[...]
    z = jnp.concatenate([g[q] for q in range(4)], axis=1) * rsq
    h = lax.dot_general(z, lw_ref[...], (((1,), (1,)), ((), ())),
                        precision=lax.Precision.HIGHEST,
                        preferred_element_type=jnp.float32)
    o_ref[...] = h + lb_ref[...]


def _tc_last(g3, dega, degb, lin_w, lin_b):
    row = lambda i: (i, 0)
    vec = lambda i: (0, 0)
    return pl.pallas_call(
        _last_body,
        grid=(NB,),
        in_specs=[
            pl.BlockSpec((4, R_BLK, F_QTR), lambda i: (0, i, 0)),
            pl.BlockSpec((R_BLK, 1), row),
            pl.BlockSpec((R_BLK, 1), row),
            pl.BlockSpec((64, 256), vec),
            pl.BlockSpec((1, 64), vec),
        ],
        out_specs=pl.BlockSpec((R_BLK, 64), row),
        out_shape=jax.ShapeDtypeStruct((N_NODES, 64), jnp.float32),
    )(g3, dega, degb, lin_w, lin_b[None, :])


def _pad_edges(src, dst, ew, granule):
    e = src.shape[0]
    e_pad = ((e + granule - 1) // granule) * granule
    pad = e_pad - e
    if pad:
        fill = (jnp.arange(pad, dtype=jnp.int32) * 37) % N_NODES
        src = jnp.concatenate([src, fill])
        dst = jnp.concatenate([dst, fill])
        ew = jnp.concatenate([ew, jnp.zeros((pad,), ew.dtype)])
    return src, dst, ew, e_pad


def kernel(x, edge_index, edge_weight, W1, b1, W2, b2, lin_w, lin_b,
           bn1_w, bn1_b, bn1_rm, bn1_rv, bn2_w, bn2_b, bn2_rm, bn2_rv):
    src = edge_index[0].astype(jnp.int32)
    dst = edge_index[1].astype(jnp.int32)
    ew = edge_weight.astype(jnp.float32)

    sD, dD, wD, epD = _pad_edges(src, dst, ew, NC * NS * 256)
    cD = epD // (NC * NS * 128)
    dstD = dD.reshape(NC, NS, cD, 128)
    ewD = wD.reshape(NC, NS, cD, 128)

    sA, dA, wA, epA = _pad_edges(src, dst, ew, NS * CHK * 2 * WIN)
    cA = epA // (NS * CHK)
    src3 = sA.reshape(NS, cA, CHK)
    dst3 = dA.reshape(NS, cA, CHK)
    ew3 = wA.reshape(NS, cA, CHK)

    degp = _make_deg(cD)(dstD, ewD)
    dega = degp[:N_NODES, None]
    degb = degp[N_PAD:N_PAD + N_NODES, None]

    agg = _make_agg(cA)

    g1 = _tc_first(x, W1, dega, degb)

    eye = jnp.eye(W2.shape[0], dtype=jnp.float32)
    xs = (jnp.stack([W2, eye]), jnp.stack([b1, b2]),
          jnp.stack([bn1_w, bn2_w]), jnp.stack([bn1_b, bn2_b]),
          jnp.stack([bn1_rm, bn2_rm]), jnp.stack([bn1_rv, bn2_rv]))

    def body(g, x_l):
        Wl, bl, bwl, bbl, brml, brvl = x_l
        accp = agg(g, src3, dst3, ew3)
        g_next = _tc_mid(accp, g, dega, degb, bl, bwl, bbl, brml, brvl, Wl)
        return g_next, None

    gz, _ = lax.scan(body, g1, xs)
    return _tc_last(gz, dega, degb, lin_w, lin_b)
